# Initial kernel scaffold; baseline (speedup 1.0000x reference)
import jax
import jax.numpy as jnp
from jax.experimental import pallas as pl


def kernel(x, edge_index, cj, ci, W):
    raise NotImplementedError("write your pallas kernel here")


# speedup vs baseline: 5.3779x; 5.3779x over previous
import functools

import jax
import jax.numpy as jnp
from jax import lax
from jax.experimental import pallas as pl
from jax.experimental.pallas import tpu as pltpu
from jax.experimental.pallas import tpu_sc as plsc

N_CORES = 2
N_SUBCORES = 16
N_WORKERS = N_CORES * N_SUBCORES
LANES = 16

CHUNK = 80
ROWBLK = 80


def _feat_body(x_ref, w_ref, cj_ref, o_ref):
    o_ref[...] = (
        jnp.dot(x_ref[...], w_ref[...], preferred_element_type=jnp.float32)
        * cj_ref[...]
    )


def _tc_feat(x, W, cj):
    n, d_in = x.shape
    d_out = W.shape[1]
    blk = 1000
    return pl.pallas_call(
        _feat_body,
        grid=(n // blk,),
        in_specs=[
            pl.BlockSpec((blk, d_in), lambda i: (i, 0)),
            pl.BlockSpec((d_in, d_out), lambda i: (0, 0)),
            pl.BlockSpec((blk, 1), lambda i: (i, 0)),
        ],
        out_specs=pl.BlockSpec((blk, d_out), lambda i: (i, 0)),
        out_shape=jax.ShapeDtypeStruct((n, d_out), jnp.float32),
    )(x, W, cj)


def _combine_body(p_ref, ci_ref, o_ref):
    o_ref[...] = (p_ref[0] + p_ref[1]) * ci_ref[...]


def _tc_combine(partial, ci):
    _, n, d = partial.shape
    blk = 1000
    return pl.pallas_call(
        _combine_body,
        grid=(n // blk,),
        in_specs=[
            pl.BlockSpec((2, blk, d), lambda i: (0, i, 0)),
            pl.BlockSpec((blk, 1), lambda i: (i, 0)),
        ],
        out_specs=pl.BlockSpec((blk, d), lambda i: (i, 0)),
        out_shape=jax.ShapeDtypeStruct((n, d), jnp.float32),
    )(partial, ci)


def _sc_scatter(feat, src, dst):
    n, d = feat.shape
    e = src.shape[0]
    e_per_w = e // N_WORKERS
    n_chunks = e_per_w // CHUNK
    n_rowblks = n // ROWBLK
    rowblks_per_tile = -(-n_rowblks // N_SUBCORES)

    mesh = plsc.VectorSubcoreMesh(core_axis_name="c", subcore_axis_name="s")

    @functools.partial(
        pl.kernel,
        mesh=mesh,
        out_type=jax.ShapeDtypeStruct((N_CORES, n, d), jnp.float32),
        scratch_types=[
            pltpu.VMEM_SHARED((n, d), jnp.float32),
            pltpu.VMEM((CHUNK,), jnp.int32),
            pltpu.VMEM((CHUNK,), jnp.int32),
            pltpu.VMEM((CHUNK, d), jnp.float32),
            pltpu.VMEM((ROWBLK, d), jnp.float32),
            pltpu.SemaphoreType.DMA,
        ],
    )
    def k(feat_hbm, src_hbm, dst_hbm, out_hbm, accum, src_v, dst_v, rows_v,
          zero_v, sem):
        cid = lax.axis_index("c")
        sid = lax.axis_index("s")
        wid = cid * N_SUBCORES + sid

        @pl.loop(0, ROWBLK)
        def _(i):
            @pl.loop(0, d, step=LANES)
            def _(j):
                zero_v[i, pl.ds(j, LANES)] = jnp.zeros((LANES,), jnp.float32)

        @pl.loop(0, rowblks_per_tile)
        def _(i):
            b = sid + i * N_SUBCORES

            @pl.when(b < n_rowblks)
            def _():
                pltpu.sync_copy(zero_v, accum.at[pl.ds(b * ROWBLK, ROWBLK)])

        plsc.subcore_barrier()

        base = wid * e_per_w

        @pl.loop(0, n_chunks)
        def _(ch):
            off = base + ch * CHUNK
            pltpu.sync_copy(src_hbm.at[pl.ds(off, CHUNK)], src_v)
            pltpu.sync_copy(dst_hbm.at[pl.ds(off, CHUNK)], dst_v)
            pltpu.async_copy(feat_hbm.at[src_v], rows_v, sem).wait()
            pltpu.sync_copy(rows_v, accum.at[dst_v], add=True)

        plsc.subcore_barrier()

        @pl.loop(0, rowblks_per_tile)
        def _(i):
            b = sid + i * N_SUBCORES

            @pl.when(b < n_rowblks)
            def _():
                pltpu.sync_copy(
                    accum.at[pl.ds(b * ROWBLK, ROWBLK)],
                    out_hbm.at[cid, pl.ds(b * ROWBLK, ROWBLK)],
                )

    return k(feat, src, dst)


def kernel(x, edge_index, cj, ci, W):
    src = edge_index[0].astype(jnp.int32)
    dst = edge_index[1].astype(jnp.int32)
    feat = _tc_feat(x, W, cj)
    partial = _sc_scatter(feat, src, dst)
    return _tc_combine(partial, ci)


# speedup vs baseline: 11.5255x; 2.1431x over previous
import functools

import jax
import jax.numpy as jnp
from jax import lax
from jax.experimental import pallas as pl
from jax.experimental.pallas import tpu as pltpu
from jax.experimental.pallas import tpu_sc as plsc

N_CORES = 2
N_SUBCORES = 16
N_WORKERS = N_CORES * N_SUBCORES
LANES = 16

CHUNK = 80
ROWBLK = 80


def _feat_body(x_ref, w_ref, cj_ref, o_ref):
    o_ref[...] = (
        jnp.dot(x_ref[...], w_ref[...], preferred_element_type=jnp.float32)
        * cj_ref[...]
    )


def _tc_feat(x, W, cj):
    n, d_in = x.shape
    d_out = W.shape[1]
    blk = 1000
    return pl.pallas_call(
        _feat_body,
        grid=(n // blk,),
        in_specs=[
            pl.BlockSpec((blk, d_in), lambda i: (i, 0)),
            pl.BlockSpec((d_in, d_out), lambda i: (0, 0)),
            pl.BlockSpec((blk, 1), lambda i: (i, 0)),
        ],
        out_specs=pl.BlockSpec((blk, d_out), lambda i: (i, 0)),
        out_shape=jax.ShapeDtypeStruct((n, d_out), jnp.float32),
    )(x, W, cj)


def _combine_body(p_ref, ci_ref, o_ref):
    o_ref[...] = (p_ref[0] + p_ref[1]) * ci_ref[...]


def _tc_combine(partial, ci):
    _, n, d = partial.shape
    blk = 1000
    return pl.pallas_call(
        _combine_body,
        grid=(n // blk,),
        in_specs=[
            pl.BlockSpec((2, blk, d), lambda i: (0, i, 0)),
            pl.BlockSpec((blk, 1), lambda i: (i, 0)),
        ],
        out_specs=pl.BlockSpec((blk, d), lambda i: (i, 0)),
        out_shape=jax.ShapeDtypeStruct((n, d), jnp.float32),
    )(partial, ci)


NBUF = 4


def _sc_scatter(feat, src3, dst3):
    n, d = feat.shape
    n_w, n_chunks, chunk = src3.shape
    n_rowblks = n // ROWBLK
    rowblks_per_tile = -(-n_rowblks // N_SUBCORES)
    n_full = n_chunks // NBUF
    rem = n_chunks % NBUF

    mesh = plsc.VectorSubcoreMesh(core_axis_name="c", subcore_axis_name="s")

    row_bufs = [pltpu.VMEM((chunk, d), jnp.float32) for _ in range(NBUF)]
    sidx_bufs = [pltpu.VMEM((2, chunk), jnp.int32) for _ in range(NBUF)]
    didx_bufs = [pltpu.VMEM((2, chunk), jnp.int32) for _ in range(NBUF)]
    sems = [pltpu.SemaphoreType.DMA for _ in range(3 * NBUF)]

    @functools.partial(
        pl.kernel,
        mesh=mesh,
        out_type=jax.ShapeDtypeStruct((N_CORES, n, d), jnp.float32),
        scratch_types=[pltpu.VMEM_SHARED((n, d), jnp.float32)]
        + row_bufs
        + sidx_bufs
        + didx_bufs
        + sems,
    )
    def k(feat_hbm, src_hbm, dst_hbm, out_hbm, accum, *rest):
        rows = rest[:NBUF]
        sidx = rest[NBUF : 2 * NBUF]
        didx = rest[2 * NBUF : 3 * NBUF]
        isem = rest[3 * NBUF : 4 * NBUF]
        gsem = rest[4 * NBUF : 5 * NBUF]
        ssem = rest[5 * NBUF : 6 * NBUF]

        cid = lax.axis_index("c")
        sid = lax.axis_index("s")
        wid = cid * N_SUBCORES + sid

        def istart(ch, b, p):
            pltpu.async_copy(src_hbm.at[wid, ch], sidx[b].at[p], isem[b])
            pltpu.async_copy(dst_hbm.at[wid, ch], didx[b].at[p], isem[b])

        def iwait(ch, b, p):
            pltpu.make_async_copy(src_hbm.at[wid, ch], sidx[b].at[p],
                                  isem[b]).wait()
            pltpu.make_async_copy(dst_hbm.at[wid, ch], didx[b].at[p],
                                  isem[b]).wait()

        def gstart(b, p):
            pltpu.async_copy(feat_hbm.at[sidx[b].at[p]], rows[b], gsem[b])

        def gwait(b, p):
            pltpu.make_async_copy(feat_hbm.at[sidx[b].at[p]], rows[b],
                                  gsem[b]).wait()

        def sstart(b, p):
            pltpu.async_copy(rows[b], accum.at[didx[b].at[p]], ssem[b],
                             add=True)

        def swait(b, p):
            pltpu.make_async_copy(rows[b], accum.at[didx[b].at[p]],
                                  ssem[b]).wait()

        for b in range(NBUF):
            istart(b, b, 0)

        @pl.loop(0, ROWBLK)
        def _(i):
            @pl.loop(0, d, step=LANES)
            def _(j):
                rows[0][i, pl.ds(j, LANES)] = jnp.zeros((LANES,), jnp.float32)

        @pl.loop(0, rowblks_per_tile)
        def _(i):
            rb = sid + i * N_SUBCORES

            @pl.when(rb < n_rowblks)
            def _():
                pltpu.sync_copy(rows[0], accum.at[pl.ds(rb * ROWBLK, ROWBLK)])

        for b in range(NBUF):
            iwait(b, b, 0)
            gstart(b, 0)

        plsc.subcore_barrier()

        @pl.loop(0, n_full)
        def _(g):
            c0 = g * NBUF
            p = lax.rem(g, 2)
            q = 1 - p
            for b in range(NBUF):
                gwait(b, p)
                sstart(b, p)
                nch = c0 + b + NBUF

                @pl.when(nch < n_chunks)
                def _():
                    istart(nch, b, q)

            for b in range(NBUF):
                swait(b, p)
                nch = c0 + b + NBUF

                @pl.when(nch < n_chunks)
                def _():
                    iwait(nch, b, q)
                    gstart(b, q)

        p_drain = n_full % 2
        for b in range(rem):
            gwait(b, p_drain)
            sstart(b, p_drain)
        for b in range(rem):
            swait(b, p_drain)

        plsc.subcore_barrier()

        @pl.loop(0, rowblks_per_tile)
        def _(i):
            rb = sid + i * N_SUBCORES

            @pl.when(rb < n_rowblks)
            def _():
                pltpu.sync_copy(
                    accum.at[pl.ds(rb * ROWBLK, ROWBLK)],
                    out_hbm.at[cid, pl.ds(rb * ROWBLK, ROWBLK)],
                )

    return k(feat, src3, dst3)


def kernel(x, edge_index, cj, ci, W):
    e = edge_index.shape[1]
    e_per_w = e // N_WORKERS
    n_chunks = e_per_w // CHUNK
    src3 = edge_index[0].astype(jnp.int32).reshape(N_WORKERS, n_chunks, CHUNK)
    dst3 = edge_index[1].astype(jnp.int32).reshape(N_WORKERS, n_chunks, CHUNK)
    feat = _tc_feat(x, W, cj)
    partial = _sc_scatter(feat, src3, dst3)
    return _tc_combine(partial, ci)


# speedup vs baseline: 11.6751x; 1.0130x over previous
import functools

import jax
import jax.numpy as jnp
from jax import lax
from jax.experimental import pallas as pl
from jax.experimental.pallas import tpu as pltpu
from jax.experimental.pallas import tpu_sc as plsc

N_CORES = 2
N_SUBCORES = 16
N_WORKERS = N_CORES * N_SUBCORES
LANES = 16

CHUNK = 80
ROWBLK = 80


def _feat_body(x_ref, w_ref, cj_ref, o_ref):
    o_ref[...] = (
        jnp.dot(x_ref[...], w_ref[...], preferred_element_type=jnp.float32)
        * cj_ref[...]
    )


def _tc_feat(x, W, cj):
    n, d_in = x.shape
    d_out = W.shape[1]
    blk = 1000
    return pl.pallas_call(
        _feat_body,
        grid=(n // blk,),
        in_specs=[
            pl.BlockSpec((blk, d_in), lambda i: (i, 0)),
            pl.BlockSpec((d_in, d_out), lambda i: (0, 0)),
            pl.BlockSpec((blk, 1), lambda i: (i, 0)),
        ],
        out_specs=pl.BlockSpec((blk, d_out), lambda i: (i, 0)),
        out_shape=jax.ShapeDtypeStruct((n, d_out), jnp.float32),
    )(x, W, cj)


def _combine_body(p_ref, ci_ref, o_ref):
    o_ref[...] = (p_ref[0] + p_ref[1]) * ci_ref[...]


def _tc_combine(partial, ci):
    _, n, d = partial.shape
    blk = 1000
    return pl.pallas_call(
        _combine_body,
        grid=(n // blk,),
        in_specs=[
            pl.BlockSpec((2, blk, d), lambda i: (0, i, 0)),
            pl.BlockSpec((blk, 1), lambda i: (i, 0)),
        ],
        out_specs=pl.BlockSpec((blk, d), lambda i: (i, 0)),
        out_shape=jax.ShapeDtypeStruct((n, d), jnp.float32),
    )(partial, ci)


NBUF = 4


def _sc_scatter(feat, src3, dst3):
    n, d = feat.shape
    n_w, n_chunks, chunk = src3.shape
    n_rowblks = n // ROWBLK
    rowblks_per_tile = -(-n_rowblks // N_SUBCORES)
    n_full = n_chunks // NBUF
    rem = n_chunks % NBUF

    mesh = plsc.VectorSubcoreMesh(core_axis_name="c", subcore_axis_name="s")

    row_bufs = [pltpu.VMEM((chunk, d), jnp.float32) for _ in range(NBUF)]
    sidx_bufs = [pltpu.VMEM((2, chunk), jnp.int32) for _ in range(NBUF)]
    didx_bufs = [pltpu.VMEM((2, chunk), jnp.int32) for _ in range(NBUF)]
    sems = [pltpu.SemaphoreType.DMA for _ in range(3 * NBUF + 1)]

    @functools.partial(
        pl.kernel,
        mesh=mesh,
        out_type=jax.ShapeDtypeStruct((N_CORES, n, d), jnp.float32),
        scratch_types=[pltpu.VMEM_SHARED((n, d), jnp.float32)]
        + row_bufs
        + sidx_bufs
        + didx_bufs
        + sems,
    )
    def k(feat_hbm, src_hbm, dst_hbm, out_hbm, accum, *rest):
        rows = rest[:NBUF]
        sidx = rest[NBUF : 2 * NBUF]
        didx = rest[2 * NBUF : 3 * NBUF]
        isem = rest[3 * NBUF : 4 * NBUF]
        gsem = rest[4 * NBUF : 5 * NBUF]
        ssem = rest[5 * NBUF : 6 * NBUF]
        zsem = rest[6 * NBUF]

        cid = lax.axis_index("c")
        sid = lax.axis_index("s")
        wid = cid * N_SUBCORES + sid

        def istart(ch, b, p):
            pltpu.async_copy(src_hbm.at[wid, ch], sidx[b].at[p], isem[b])
            pltpu.async_copy(dst_hbm.at[wid, ch], didx[b].at[p], isem[b])

        def iwait(ch, b, p):
            pltpu.make_async_copy(src_hbm.at[wid, ch], sidx[b].at[p],
                                  isem[b]).wait()
            pltpu.make_async_copy(dst_hbm.at[wid, ch], didx[b].at[p],
                                  isem[b]).wait()

        def gstart(b, p):
            pltpu.async_copy(feat_hbm.at[sidx[b].at[p]], rows[b], gsem[b])

        def gwait(b, p):
            pltpu.make_async_copy(feat_hbm.at[sidx[b].at[p]], rows[b],
                                  gsem[b]).wait()

        def sstart(b, p):
            pltpu.async_copy(rows[b], accum.at[didx[b].at[p]], ssem[b],
                             add=True)

        def swait(b, p):
            pltpu.make_async_copy(rows[b], accum.at[didx[b].at[p]],
                                  ssem[b]).wait()

        for b in range(NBUF):
            istart(b, b, 0)

        @pl.loop(0, ROWBLK)
        def _(i):
            @pl.loop(0, d, step=LANES)
            def _(j):
                rows[0][i, pl.ds(j, LANES)] = jnp.zeros((LANES,), jnp.float32)

        @pl.loop(0, rowblks_per_tile)
        def _(i):
            rb = sid + i * N_SUBCORES

            @pl.when(rb < n_rowblks)
            def _():
                pltpu.async_copy(rows[0], accum.at[pl.ds(rb * ROWBLK, ROWBLK)],
                                 zsem)

        for b in range(1, NBUF):
            iwait(b, b, 0)
            gstart(b, 0)

        @pl.loop(0, rowblks_per_tile)
        def _(i):
            rb = sid + i * N_SUBCORES

            @pl.when(rb < n_rowblks)
            def _():
                pltpu.make_async_copy(
                    rows[0], accum.at[pl.ds(rb * ROWBLK, ROWBLK)], zsem
                ).wait()

        iwait(0, 0, 0)
        gstart(0, 0)

        plsc.subcore_barrier()

        @pl.loop(0, n_full)
        def _(g):
            c0 = g * NBUF
            p = lax.rem(g, 2)
            q = 1 - p
            for b in range(NBUF):
                gwait(b, p)
                sstart(b, p)
                nch = c0 + b + NBUF

                @pl.when(nch < n_chunks)
                def _():
                    istart(nch, b, q)

            for b in range(NBUF):
                swait(b, p)
                nch = c0 + b + NBUF

                @pl.when(nch < n_chunks)
                def _():
                    iwait(nch, b, q)
                    gstart(b, q)

        p_drain = n_full % 2
        for b in range(rem):
            gwait(b, p_drain)
            sstart(b, p_drain)
        for b in range(rem):
            swait(b, p_drain)

        plsc.subcore_barrier()

        @pl.loop(0, rowblks_per_tile)
        def _(i):
            rb = sid + i * N_SUBCORES

            @pl.when(rb < n_rowblks)
            def _():
                pltpu.sync_copy(
                    accum.at[pl.ds(rb * ROWBLK, ROWBLK)],
                    out_hbm.at[cid, pl.ds(rb * ROWBLK, ROWBLK)],
                )

    return k(feat, src3, dst3)


def kernel(x, edge_index, cj, ci, W):
    e = edge_index.shape[1]
    e_per_w = e // N_WORKERS
    n_chunks = e_per_w // CHUNK
    src3 = edge_index[0].astype(jnp.int32).reshape(N_WORKERS, n_chunks, CHUNK)
    dst3 = edge_index[1].astype(jnp.int32).reshape(N_WORKERS, n_chunks, CHUNK)
    feat = _tc_feat(x, W, cj)
    partial = _sc_scatter(feat, src3, dst3)
    return _tc_combine(partial, ci)


# speedup vs baseline: 11.9887x; 1.0269x over previous
import functools

import jax
import jax.numpy as jnp
from jax import lax
from jax.experimental import pallas as pl
from jax.experimental.pallas import tpu as pltpu
from jax.experimental.pallas import tpu_sc as plsc

N_CORES = 2
N_SUBCORES = 16
N_WORKERS = N_CORES * N_SUBCORES
LANES = 16

CHUNK = 80
ROWBLK = 80


def _feat_body(x_ref, w_ref, cj_ref, o_ref):
    o_ref[...] = (
        jnp.dot(x_ref[...], w_ref[...], preferred_element_type=jnp.float32)
        * cj_ref[...]
    )


def _tc_feat(x, W, cj):
    n, d_in = x.shape
    d_out = W.shape[1]
    blk = 2000
    return pl.pallas_call(
        _feat_body,
        grid=(n // blk,),
        in_specs=[
            pl.BlockSpec((blk, d_in), lambda i: (i, 0)),
            pl.BlockSpec((d_in, d_out), lambda i: (0, 0)),
            pl.BlockSpec((blk, 1), lambda i: (i, 0)),
        ],
        out_specs=pl.BlockSpec((blk, d_out), lambda i: (i, 0)),
        out_shape=jax.ShapeDtypeStruct((n, d_out), jnp.float32),
    )(x, W, cj)


def _combine_body(p_ref, ci_ref, o_ref):
    o_ref[...] = (p_ref[0] + p_ref[1]) * ci_ref[...]


def _tc_combine(partial, ci):
    _, n, d = partial.shape
    blk = 2000
    return pl.pallas_call(
        _combine_body,
        grid=(n // blk,),
        in_specs=[
            pl.BlockSpec((2, blk, d), lambda i: (0, i, 0)),
            pl.BlockSpec((blk, 1), lambda i: (i, 0)),
        ],
        out_specs=pl.BlockSpec((blk, d), lambda i: (i, 0)),
        out_shape=jax.ShapeDtypeStruct((n, d), jnp.float32),
    )(partial, ci)


NBUF = 4


def _sc_scatter(feat, src3, dst3):
    n, d = feat.shape
    n_w, n_chunks, chunk = src3.shape
    n_rowblks = n // ROWBLK
    rowblks_per_tile = -(-n_rowblks // N_SUBCORES)
    n_full = n_chunks // NBUF
    rem = n_chunks % NBUF

    mesh = plsc.VectorSubcoreMesh(core_axis_name="c", subcore_axis_name="s")

    row_bufs = [pltpu.VMEM((chunk, d), jnp.float32) for _ in range(NBUF)]
    sidx_bufs = [pltpu.VMEM((2, chunk), jnp.int32) for _ in range(NBUF)]
    didx_bufs = [pltpu.VMEM((2, chunk), jnp.int32) for _ in range(NBUF)]
    sems = [pltpu.SemaphoreType.DMA for _ in range(3 * NBUF + 1)]

    @functools.partial(
        pl.kernel,
        mesh=mesh,
        out_type=jax.ShapeDtypeStruct((N_CORES, n, d), jnp.float32),
        scratch_types=[pltpu.VMEM_SHARED((n, d), jnp.float32)]
        + row_bufs
        + sidx_bufs
        + didx_bufs
        + sems,
    )
    def k(feat_hbm, src_hbm, dst_hbm, out_hbm, accum, *rest):
        rows = rest[:NBUF]
        sidx = rest[NBUF : 2 * NBUF]
        didx = rest[2 * NBUF : 3 * NBUF]
        isem = rest[3 * NBUF : 4 * NBUF]
        gsem = rest[4 * NBUF : 5 * NBUF]
        ssem = rest[5 * NBUF : 6 * NBUF]
        zsem = rest[6 * NBUF]

        cid = lax.axis_index("c")
        sid = lax.axis_index("s")
        wid = cid * N_SUBCORES + sid

        def istart(ch, b, p):
            pltpu.async_copy(src_hbm.at[wid, ch], sidx[b].at[p], isem[b])
            pltpu.async_copy(dst_hbm.at[wid, ch], didx[b].at[p], isem[b])

        def iwait(ch, b, p):
            pltpu.make_async_copy(src_hbm.at[wid, ch], sidx[b].at[p],
                                  isem[b]).wait()
            pltpu.make_async_copy(dst_hbm.at[wid, ch], didx[b].at[p],
                                  isem[b]).wait()

        def gstart(b, p):
            pltpu.async_copy(feat_hbm.at[sidx[b].at[p]], rows[b], gsem[b])

        def gwait(b, p):
            pltpu.make_async_copy(feat_hbm.at[sidx[b].at[p]], rows[b],
                                  gsem[b]).wait()

        def sstart(b, p):
            pltpu.async_copy(rows[b], accum.at[didx[b].at[p]], ssem[b],
                             add=True)

        def swait(b, p):
            pltpu.make_async_copy(rows[b], accum.at[didx[b].at[p]],
                                  ssem[b]).wait()

        for b in range(NBUF):
            istart(b, b, 0)

        @pl.loop(0, ROWBLK)
        def _(i):
            @pl.loop(0, d, step=LANES)
            def _(j):
                rows[0][i, pl.ds(j, LANES)] = jnp.zeros((LANES,), jnp.float32)

        @pl.loop(0, rowblks_per_tile)
        def _(i):
            rb = sid + i * N_SUBCORES

            @pl.when(rb < n_rowblks)
            def _():
                pltpu.async_copy(rows[0], accum.at[pl.ds(rb * ROWBLK, ROWBLK)],
                                 zsem)

        for b in range(1, NBUF):
            iwait(b, b, 0)
            gstart(b, 0)

        @pl.loop(0, rowblks_per_tile)
        def _(i):
            rb = sid + i * N_SUBCORES

            @pl.when(rb < n_rowblks)
            def _():
                pltpu.make_async_copy(
                    rows[0], accum.at[pl.ds(rb * ROWBLK, ROWBLK)], zsem
                ).wait()

        iwait(0, 0, 0)
        gstart(0, 0)

        plsc.subcore_barrier()

        @pl.loop(0, n_full)
        def _(g):
            c0 = g * NBUF
            p = lax.rem(g, 2)
            q = 1 - p
            for b in range(NBUF):
                gwait(b, p)
                sstart(b, p)
                nch = c0 + b + NBUF

                @pl.when(nch < n_chunks)
                def _():
                    istart(nch, b, q)

            for b in range(NBUF):
                swait(b, p)
                nch = c0 + b + NBUF

                @pl.when(nch < n_chunks)
                def _():
                    iwait(nch, b, q)
                    gstart(b, q)

        p_drain = n_full % 2
        for b in range(rem):
            gwait(b, p_drain)
            sstart(b, p_drain)
        for b in range(rem):
            swait(b, p_drain)

        plsc.subcore_barrier()

        @pl.loop(0, rowblks_per_tile)
        def _(i):
            rb = sid + i * N_SUBCORES

            @pl.when(rb < n_rowblks)
            def _():
                pltpu.sync_copy(
                    accum.at[pl.ds(rb * ROWBLK, ROWBLK)],
                    out_hbm.at[cid, pl.ds(rb * ROWBLK, ROWBLK)],
                )

    return k(feat, src3, dst3)


def kernel(x, edge_index, cj, ci, W):
    e = edge_index.shape[1]
    e_per_w = e // N_WORKERS
    n_chunks = e_per_w // CHUNK
    src3 = edge_index[0].astype(jnp.int32).reshape(N_WORKERS, n_chunks, CHUNK)
    dst3 = edge_index[1].astype(jnp.int32).reshape(N_WORKERS, n_chunks, CHUNK)
    feat = _tc_feat(x, W, cj)
    partial = _sc_scatter(feat, src3, dst3)
    return _tc_combine(partial, ci)


# speedup vs baseline: 12.2073x; 1.0182x over previous
import functools

import jax
import jax.numpy as jnp
from jax import lax
from jax.experimental import pallas as pl
from jax.experimental.pallas import tpu as pltpu
from jax.experimental.pallas import tpu_sc as plsc

N_CORES = 2
N_SUBCORES = 16
N_WORKERS = N_CORES * N_SUBCORES
LANES = 16

CHUNK = 80
ROWBLK = 80


def _feat_body(x_ref, w_ref, cj_ref, o_ref):
    o_ref[...] = (
        jnp.dot(x_ref[...], w_ref[...], preferred_element_type=jnp.float32)
        * cj_ref[...]
    )


def _tc_feat(x, W, cj):
    n, d_in = x.shape
    d_out = W.shape[1]
    blk = 5000
    return pl.pallas_call(
        _feat_body,
        grid=(n // blk,),
        in_specs=[
            pl.BlockSpec((blk, d_in), lambda i: (i, 0)),
            pl.BlockSpec((d_in, d_out), lambda i: (0, 0)),
            pl.BlockSpec((blk, 1), lambda i: (i, 0)),
        ],
        out_specs=pl.BlockSpec((blk, d_out), lambda i: (i, 0)),
        out_shape=jax.ShapeDtypeStruct((n, d_out), jnp.float32),
    )(x, W, cj)


def _combine_body(p_ref, ci_ref, o_ref):
    o_ref[...] = (p_ref[0] + p_ref[1]) * ci_ref[...]


def _tc_combine(partial, ci):
    _, n, d = partial.shape
    blk = 5000
    return pl.pallas_call(
        _combine_body,
        grid=(n // blk,),
        in_specs=[
            pl.BlockSpec((2, blk, d), lambda i: (0, i, 0)),
            pl.BlockSpec((blk, 1), lambda i: (i, 0)),
        ],
        out_specs=pl.BlockSpec((blk, d), lambda i: (i, 0)),
        out_shape=jax.ShapeDtypeStruct((n, d), jnp.float32),
    )(partial, ci)


NBUF = 4


def _sc_scatter(feat, src3, dst3):
    n, d = feat.shape
    n_w, n_chunks, chunk = src3.shape
    n_rowblks = n // ROWBLK
    rowblks_per_tile = -(-n_rowblks // N_SUBCORES)
    n_full = n_chunks // NBUF
    rem = n_chunks % NBUF

    mesh = plsc.VectorSubcoreMesh(core_axis_name="c", subcore_axis_name="s")

    row_bufs = [pltpu.VMEM((chunk, d), jnp.float32) for _ in range(NBUF)]
    sidx_bufs = [pltpu.VMEM((2, chunk), jnp.int32) for _ in range(NBUF)]
    didx_bufs = [pltpu.VMEM((2, chunk), jnp.int32) for _ in range(NBUF)]
    sems = [pltpu.SemaphoreType.DMA for _ in range(3 * NBUF + 1)]

    @functools.partial(
        pl.kernel,
        mesh=mesh,
        out_type=jax.ShapeDtypeStruct((N_CORES, n, d), jnp.float32),
        scratch_types=[pltpu.VMEM_SHARED((n, d), jnp.float32)]
        + row_bufs
        + sidx_bufs
        + didx_bufs
        + sems,
    )
    def k(feat_hbm, src_hbm, dst_hbm, out_hbm, accum, *rest):
        rows = rest[:NBUF]
        sidx = rest[NBUF : 2 * NBUF]
        didx = rest[2 * NBUF : 3 * NBUF]
        isem = rest[3 * NBUF : 4 * NBUF]
        gsem = rest[4 * NBUF : 5 * NBUF]
        ssem = rest[5 * NBUF : 6 * NBUF]
        zsem = rest[6 * NBUF]

        cid = lax.axis_index("c")
        sid = lax.axis_index("s")
        wid = cid * N_SUBCORES + sid

        def istart(ch, b, p):
            pltpu.async_copy(src_hbm.at[wid, ch], sidx[b].at[p], isem[b])
            pltpu.async_copy(dst_hbm.at[wid, ch], didx[b].at[p], isem[b])

        def iwait(ch, b, p):
            pltpu.make_async_copy(src_hbm.at[wid, ch], sidx[b].at[p],
                                  isem[b]).wait()
            pltpu.make_async_copy(dst_hbm.at[wid, ch], didx[b].at[p],
                                  isem[b]).wait()

        def gstart(b, p):
            pltpu.async_copy(feat_hbm.at[sidx[b].at[p]], rows[b], gsem[b])

        def gwait(b, p):
            pltpu.make_async_copy(feat_hbm.at[sidx[b].at[p]], rows[b],
                                  gsem[b]).wait()

        def sstart(b, p):
            pltpu.async_copy(rows[b], accum.at[didx[b].at[p]], ssem[b],
                             add=True)

        def swait(b, p):
            pltpu.make_async_copy(rows[b], accum.at[didx[b].at[p]],
                                  ssem[b]).wait()

        for b in range(NBUF):
            istart(b, b, 0)

        @pl.loop(0, ROWBLK)
        def _(i):
            @pl.loop(0, d, step=LANES)
            def _(j):
                rows[0][i, pl.ds(j, LANES)] = jnp.zeros((LANES,), jnp.float32)

        @pl.loop(0, rowblks_per_tile)
        def _(i):
            rb = sid + i * N_SUBCORES

            @pl.when(rb < n_rowblks)
            def _():
                pltpu.async_copy(rows[0], accum.at[pl.ds(rb * ROWBLK, ROWBLK)],
                                 zsem)

        for b in range(1, NBUF):
            iwait(b, b, 0)
            gstart(b, 0)

        @pl.loop(0, rowblks_per_tile)
        def _(i):
            rb = sid + i * N_SUBCORES

            @pl.when(rb < n_rowblks)
            def _():
                pltpu.make_async_copy(
                    rows[0], accum.at[pl.ds(rb * ROWBLK, ROWBLK)], zsem
                ).wait()

        iwait(0, 0, 0)
        gstart(0, 0)

        plsc.subcore_barrier()

        @pl.loop(0, n_full)
        def _(g):
            c0 = g * NBUF
            p = lax.rem(g, 2)
            q = 1 - p
            for b in range(NBUF):
                gwait(b, p)
                sstart(b, p)
                nch = c0 + b + NBUF

                @pl.when(nch < n_chunks)
                def _():
                    istart(nch, b, q)

            for b in range(NBUF):
                swait(b, p)
                nch = c0 + b + NBUF

                @pl.when(nch < n_chunks)
                def _():
                    iwait(nch, b, q)
                    gstart(b, q)

        p_drain = n_full % 2
        for b in range(rem):
            gwait(b, p_drain)
            sstart(b, p_drain)
        for b in range(rem):
            swait(b, p_drain)

        plsc.subcore_barrier()

        @pl.loop(0, rowblks_per_tile)
        def _(i):
            rb = sid + i * N_SUBCORES

            @pl.when(rb < n_rowblks)
            def _():
                pltpu.sync_copy(
                    accum.at[pl.ds(rb * ROWBLK, ROWBLK)],
                    out_hbm.at[cid, pl.ds(rb * ROWBLK, ROWBLK)],
                )

    return k(feat, src3, dst3)


def kernel(x, edge_index, cj, ci, W):
    e = edge_index.shape[1]
    e_per_w = e // N_WORKERS
    n_chunks = e_per_w // CHUNK
    src3 = edge_index[0].astype(jnp.int32).reshape(N_WORKERS, n_chunks, CHUNK)
    dst3 = edge_index[1].astype(jnp.int32).reshape(N_WORKERS, n_chunks, CHUNK)
    feat = _tc_feat(x, W, cj)
    partial = _sc_scatter(feat, src3, dst3)
    return _tc_combine(partial, ci)
